# baseline (speedup 1.0000x reference)
import functools

import jax
import jax.numpy as jnp
from jax import lax
from jax.experimental import pallas as pl
from jax.experimental.pallas import tpu as pltpu
from jax.experimental.pallas import tpu_sc as plsc

N_NODES = 10000
N_EDGES = 160000
D_FEAT = 128
LANES = 16
NUM_CORES = 2
NUM_SUBCORES = 16
NUM_WORKERS = NUM_CORES * NUM_SUBCORES
CHUNK = N_EDGES // NUM_WORKERS
NFULL = CHUNK // LANES
ACC_WORDS = 10240
OFF_SUM_R, OFF_SUM_X, OFF_SQ_R, OFF_SQ_X = 10000, 10016, 10032, 10048
STRIPE = ACC_WORDS // NUM_SUBCORES
VRANGE = 320
SLAB = 32
NSLAB = VRANGE // SLAB


def _vx_body(nf_hbm, v_hbm, ring0_v, ring1_v, ring2_v, ring3_v, vtmp_v,
             rsem0, rsem1, rsem2, rsem3):
    c = lax.axis_index("c")
    s = lax.axis_index("s")
    w = c * NUM_SUBCORES + s
    vstart = jnp.minimum(VRANGE * w, N_NODES - VRANGE)
    lane = lax.iota(jnp.int32, LANES)
    lane128 = lane * D_FEAT
    rings = [ring0_v, ring1_v, ring2_v, ring3_v]
    rsems = [rsem0, rsem1, rsem2, rsem3]

    def _issue(j):
        return pltpu.async_copy(
            nf_hbm.at[pl.ds((vstart + SLAB * j) * D_FEAT, SLAB * D_FEAT)],
            rings[j % 4], rsems[j % 4])

    ring_cps = [_issue(j) for j in range(4)]
    for j in range(NSLAB):
        ring_cps[j % 4].wait()
        v0 = plsc.load_gather(rings[j % 4], [lane128])
        v1 = plsc.load_gather(rings[j % 4], [lane128 + LANES * D_FEAT])
        if j + 4 < NSLAB:
            ring_cps[j % 4] = _issue(j + 4)
        vtmp_v[pl.ds(SLAB * j, LANES)] = v0
        vtmp_v[pl.ds(SLAB * j + LANES, LANES)] = v1
    pltpu.sync_copy(vtmp_v, v_hbm.at[pl.ds(vstart, VRANGE)])


_vx_kernel = functools.partial(
    pl.kernel,
    out_type=jax.ShapeDtypeStruct((N_NODES,), jnp.float32),
    mesh=plsc.VectorSubcoreMesh(core_axis_name="c", subcore_axis_name="s"),
    compiler_params=pltpu.CompilerParams(needs_layout_passes=False),
    scratch_types=[
        pltpu.VMEM((SLAB * D_FEAT,), jnp.float32),
        pltpu.VMEM((SLAB * D_FEAT,), jnp.float32),
        pltpu.VMEM((SLAB * D_FEAT,), jnp.float32),
        pltpu.VMEM((SLAB * D_FEAT,), jnp.float32),
        pltpu.VMEM((VRANGE,), jnp.float32),
        pltpu.SemaphoreType.DMA,
        pltpu.SemaphoreType.DMA,
        pltpu.SemaphoreType.DMA,
        pltpu.SemaphoreType.DMA,
    ],
)(_vx_body)


def _sc_body(v_hbm, ei_hbm, lg_hbm, r_hbm, x_hbm, out_hbm,
             v_v, src_v, dst_v, lg_v, r_v, x_v,
             acc_v, stripes_v, red_v, shacc_s,
             sem0, sem1, sem2, sem3, sem4, sem5):
    c = lax.axis_index("c")
    s = lax.axis_index("s")
    w = c * NUM_SUBCORES + s
    ebase = w * CHUNK

    cps = [
        pltpu.async_copy(v_hbm, v_v, sem5),
        pltpu.async_copy(ei_hbm.at[pl.ds(ebase, CHUNK)], src_v, sem0),
        pltpu.async_copy(ei_hbm.at[pl.ds(N_EDGES + ebase, CHUNK)], dst_v,
                         sem1),
        pltpu.async_copy(lg_hbm.at[pl.ds(ebase, CHUNK)], lg_v, sem2),
        pltpu.async_copy(r_hbm.at[pl.ds(ebase, CHUNK)], r_v, sem3),
        pltpu.async_copy(x_hbm.at[pl.ds(ebase, CHUNK)], x_v, sem4),
    ]

    lane = lax.iota(jnp.int32, LANES)
    zero16 = jnp.zeros((LANES,), jnp.float32)

    def _zero(i, _):
        acc_v[pl.ds(i * LANES, LANES)] = zero16
        return 0
    lax.fori_loop(0, ACC_WORDS // LANES, _zero, 0, unroll=8)
    for cp in cps:
        cp.wait()

    def _edge_step(b, mask):
        si = src_v[pl.ds(b, LANES)]
        di = dst_v[pl.ds(b, LANES)]
        if mask is not None:
            si = jnp.where(mask, si, 0)
            di = jnp.where(mask, di, 0)
        lg = lg_v[pl.ds(b, LANES)]
        rr = r_v[pl.ds(b, LANES)]
        rx = x_v[pl.ds(b, LANES)]
        vs = plsc.load_gather(v_v, [si])
        vd = plsc.load_gather(v_v, [di])
        imp = rr + rx + jnp.float32(1e-6)
        diff = jnp.abs(vs - vd)
        wgt = diff / (imp * (jnp.float32(1.0) + jnp.exp(-lg)))
        plsc.addupdate_scatter(acc_v, [di], wgt)
        plsc.addupdate_scatter(acc_v, [si], -wgt)

    plsc.parallel_loop(0, NFULL, unroll=8)(
        lambda i: _edge_step(i * LANES, None))
    _edge_step(CHUNK - LANES, lane >= 8)

    def _kvl(i, carry):
        s_r, s_x, q_r, q_x = carry
        rr = r_v[pl.ds(i * LANES, LANES)]
        rx = x_v[pl.ds(i * LANES, LANES)]
        return (s_r + rr, s_x + rx, q_r + rr * rr, q_x + rx * rx)

    s_r, s_x, q_r, q_x = lax.fori_loop(
        0, NFULL, _kvl, (zero16, zero16, zero16, zero16), unroll=4)
    tmask = lane >= 8
    rr = jnp.where(tmask, r_v[pl.ds(CHUNK - LANES, LANES)], 0.0)
    rx = jnp.where(tmask, x_v[pl.ds(CHUNK - LANES, LANES)], 0.0)
    acc_v[pl.ds(OFF_SUM_R, LANES)] = s_r + rr
    acc_v[pl.ds(OFF_SUM_X, LANES)] = s_x + rx
    acc_v[pl.ds(OFF_SQ_R, LANES)] = q_r + rr * rr
    acc_v[pl.ds(OFF_SQ_X, LANES)] = q_x + rx * rx

    pltpu.sync_copy(acc_v, shacc_s.at[s])
    plsc.subcore_barrier()
    words = pl.ds(STRIPE * s, STRIPE)
    pltpu.sync_copy(shacc_s.at[:, words], stripes_v)

    def _acc_vec(k, _):
        sl = pl.ds(k * LANES, LANES)
        acc = stripes_v[0, sl]
        for j in range(1, NUM_SUBCORES):
            acc = acc + stripes_v[j, sl]
        red_v[sl] = acc
        return 0
    lax.fori_loop(0, STRIPE // LANES, _acc_vec, 0, unroll=2)
    pltpu.sync_copy(red_v, out_hbm.at[c, words])


_sc_kernel = functools.partial(
    pl.kernel,
    out_type=jax.ShapeDtypeStruct((NUM_CORES, ACC_WORDS), jnp.float32),
    mesh=plsc.VectorSubcoreMesh(core_axis_name="c", subcore_axis_name="s"),
    compiler_params=pltpu.CompilerParams(needs_layout_passes=False),
    scratch_types=[
        pltpu.VMEM((N_NODES,), jnp.float32),
        pltpu.VMEM((CHUNK,), jnp.int32),
        pltpu.VMEM((CHUNK,), jnp.int32),
        pltpu.VMEM((CHUNK,), jnp.float32),
        pltpu.VMEM((CHUNK,), jnp.float32),
        pltpu.VMEM((CHUNK,), jnp.float32),
        pltpu.VMEM((ACC_WORDS,), jnp.float32),
        pltpu.VMEM((NUM_SUBCORES, STRIPE), jnp.float32),
        pltpu.VMEM((STRIPE,), jnp.float32),
        pltpu.VMEM_SHARED((NUM_SUBCORES, ACC_WORDS), jnp.float32),
        pltpu.SemaphoreType.DMA,
        pltpu.SemaphoreType.DMA,
        pltpu.SemaphoreType.DMA,
        pltpu.SemaphoreType.DMA,
        pltpu.SemaphoreType.DMA,
        pltpu.SemaphoreType.DMA,
    ],
)(_sc_body)


def _finish_kernel(p_ref, o_ref):
    p = p_ref[...]
    tot = (p[0] + p[1]).reshape(ACC_WORDS // D_FEAT, D_FEAT)
    fidx = (lax.broadcasted_iota(jnp.int32, tot.shape, 0) * D_FEAT
            + lax.broadcasted_iota(jnp.int32, tot.shape, 1))
    nodes = jnp.where(fidx < N_NODES, tot, 0.0)
    kcl = jnp.sum(nodes * nodes) / jnp.float32(N_NODES)

    def _sum16(off):
        m = (fidx >= off) & (fidx < off + LANES)
        return jnp.sum(jnp.where(m, tot, 0.0))

    s_r, s_x = _sum16(OFF_SUM_R), _sum16(OFF_SUM_X)
    q_r, q_x = _sum16(OFF_SQ_R), _sum16(OFF_SQ_X)
    n = jnp.float32(N_EDGES)
    var_r = (q_r - s_r * s_r / n) / (n - 1.0)
    var_x = (q_x - s_x * s_x / n) / (n - 1.0)
    o_ref[0, 0] = kcl + 0.5 * (var_r + var_x)


def kernel(node_features, edge_index, edge_logits, edge_params):
    nf_flat = node_features.reshape(-1)
    r = edge_params[:, 0]
    x = edge_params[:, 1]
    v = _vx_kernel(nf_flat)
    partial = _sc_kernel(v, edge_index.reshape(-1), edge_logits, r, x)
    out = pl.pallas_call(
        _finish_kernel,
        out_shape=jax.ShapeDtypeStruct((1, 1), jnp.float32),
        out_specs=pl.BlockSpec(memory_space=pltpu.SMEM),
    )(partial)
    return out[0, 0]


# speedup vs baseline: 1.0025x; 1.0025x over previous
import functools

import jax
import jax.numpy as jnp
from jax import lax
from jax.experimental import pallas as pl
from jax.experimental.pallas import tpu as pltpu
from jax.experimental.pallas import tpu_sc as plsc

N_NODES = 10000
N_EDGES = 160000
D_FEAT = 128
LANES = 16
NUM_CORES = 2
NUM_SUBCORES = 16
NUM_WORKERS = NUM_CORES * NUM_SUBCORES
CHUNK = N_EDGES // NUM_WORKERS
NFULL = CHUNK // LANES
ACC_WORDS = 10240
OFF_SUM_R, OFF_SUM_X, OFF_SQ_R, OFF_SQ_X = 10000, 10016, 10032, 10048
STRIPE = ACC_WORDS // NUM_SUBCORES
VRANGE = 320
SLAB = 32
NSLAB = VRANGE // SLAB


def _vx_body(nf_hbm, v_hbm, ring0_v, ring1_v, ring2_v, ring3_v, vtmp_v,
             rsem0, rsem1, rsem2, rsem3):
    c = lax.axis_index("c")
    s = lax.axis_index("s")
    w = c * NUM_SUBCORES + s
    vstart = jnp.minimum(VRANGE * w, N_NODES - VRANGE)
    lane = lax.iota(jnp.int32, LANES)
    lane128 = lane * D_FEAT
    rings = [ring0_v, ring1_v, ring2_v, ring3_v]
    rsems = [rsem0, rsem1, rsem2, rsem3]

    def _issue(j):
        return pltpu.async_copy(
            nf_hbm.at[pl.ds((vstart + SLAB * j) * D_FEAT, SLAB * D_FEAT)],
            rings[j % 4], rsems[j % 4])

    ring_cps = [_issue(j) for j in range(4)]
    for j in range(NSLAB):
        ring_cps[j % 4].wait()
        v0 = plsc.load_gather(rings[j % 4], [lane128])
        v1 = plsc.load_gather(rings[j % 4], [lane128 + LANES * D_FEAT])
        if j + 4 < NSLAB:
            ring_cps[j % 4] = _issue(j + 4)
        vtmp_v[pl.ds(SLAB * j, LANES)] = v0
        vtmp_v[pl.ds(SLAB * j + LANES, LANES)] = v1
    pltpu.sync_copy(vtmp_v, v_hbm.at[pl.ds(vstart, VRANGE)])


_vx_kernel = functools.partial(
    pl.kernel,
    out_type=jax.ShapeDtypeStruct((N_NODES,), jnp.float32),
    mesh=plsc.VectorSubcoreMesh(core_axis_name="c", subcore_axis_name="s"),
    compiler_params=pltpu.CompilerParams(needs_layout_passes=False),
    scratch_types=[
        pltpu.VMEM((SLAB * D_FEAT,), jnp.float32),
        pltpu.VMEM((SLAB * D_FEAT,), jnp.float32),
        pltpu.VMEM((SLAB * D_FEAT,), jnp.float32),
        pltpu.VMEM((SLAB * D_FEAT,), jnp.float32),
        pltpu.VMEM((VRANGE,), jnp.float32),
        pltpu.SemaphoreType.DMA,
        pltpu.SemaphoreType.DMA,
        pltpu.SemaphoreType.DMA,
        pltpu.SemaphoreType.DMA,
    ],
)(_vx_body)


def _sc_body(v_hbm, ei_hbm, lg_hbm, r_hbm, x_hbm, out_hbm,
             v_v, src_v, dst_v, lg_v, r_v, x_v,
             acc_v, stripes_v, red_v, shacc_s,
             sem0, sem1, sem2, sem3, sem4, sem5):
    c = lax.axis_index("c")
    s = lax.axis_index("s")
    w = c * NUM_SUBCORES + s
    ebase = w * CHUNK

    cps = [
        pltpu.async_copy(v_hbm, v_v, sem5),
        pltpu.async_copy(ei_hbm.at[pl.ds(ebase, CHUNK)], src_v, sem0),
        pltpu.async_copy(ei_hbm.at[pl.ds(N_EDGES + ebase, CHUNK)], dst_v,
                         sem1),
        pltpu.async_copy(lg_hbm.at[pl.ds(ebase, CHUNK)], lg_v, sem2),
        pltpu.async_copy(r_hbm.at[pl.ds(ebase, CHUNK)], r_v, sem3),
        pltpu.async_copy(x_hbm.at[pl.ds(ebase, CHUNK)], x_v, sem4),
    ]

    lane = lax.iota(jnp.int32, LANES)
    zero16 = jnp.zeros((LANES,), jnp.float32)

    def _zero(i, _):
        acc_v[pl.ds(i * LANES, LANES)] = zero16
        return 0
    lax.fori_loop(0, ACC_WORDS // LANES, _zero, 0, unroll=8)
    for cp in cps:
        cp.wait()

    def _edge_step(b, mask):
        si = src_v[pl.ds(b, LANES)]
        di = dst_v[pl.ds(b, LANES)]
        if mask is not None:
            si = jnp.where(mask, si, 0)
            di = jnp.where(mask, di, 0)
        lg = lg_v[pl.ds(b, LANES)]
        rr = r_v[pl.ds(b, LANES)]
        rx = x_v[pl.ds(b, LANES)]
        vs = plsc.load_gather(v_v, [si])
        vd = plsc.load_gather(v_v, [di])
        imp = rr + rx + jnp.float32(1e-6)
        diff = jnp.abs(vs - vd)
        wgt = diff / (imp * (jnp.float32(1.0) + jnp.exp(-lg)))
        plsc.addupdate_scatter(acc_v, [di], wgt)
        plsc.addupdate_scatter(acc_v, [si], -wgt)

    plsc.parallel_loop(0, NFULL, unroll=4)(
        lambda i: _edge_step(i * LANES, None))
    _edge_step(CHUNK - LANES, lane >= 8)

    def _kvl(i, carry):
        s_r, s_x, q_r, q_x = carry
        rr = r_v[pl.ds(i * LANES, LANES)]
        rx = x_v[pl.ds(i * LANES, LANES)]
        return (s_r + rr, s_x + rx, q_r + rr * rr, q_x + rx * rx)

    s_r, s_x, q_r, q_x = lax.fori_loop(
        0, NFULL, _kvl, (zero16, zero16, zero16, zero16), unroll=4)
    tmask = lane >= 8
    rr = jnp.where(tmask, r_v[pl.ds(CHUNK - LANES, LANES)], 0.0)
    rx = jnp.where(tmask, x_v[pl.ds(CHUNK - LANES, LANES)], 0.0)
    acc_v[pl.ds(OFF_SUM_R, LANES)] = s_r + rr
    acc_v[pl.ds(OFF_SUM_X, LANES)] = s_x + rx
    acc_v[pl.ds(OFF_SQ_R, LANES)] = q_r + rr * rr
    acc_v[pl.ds(OFF_SQ_X, LANES)] = q_x + rx * rx

    pltpu.sync_copy(acc_v, shacc_s.at[s])
    plsc.subcore_barrier()
    words = pl.ds(STRIPE * s, STRIPE)
    pltpu.sync_copy(shacc_s.at[:, words], stripes_v)

    def _acc_vec(k, _):
        sl = pl.ds(k * LANES, LANES)
        acc = stripes_v[0, sl]
        for j in range(1, NUM_SUBCORES):
            acc = acc + stripes_v[j, sl]
        red_v[sl] = acc
        return 0
    lax.fori_loop(0, STRIPE // LANES, _acc_vec, 0, unroll=2)
    pltpu.sync_copy(red_v, out_hbm.at[c, words])


_sc_kernel = functools.partial(
    pl.kernel,
    out_type=jax.ShapeDtypeStruct((NUM_CORES, ACC_WORDS), jnp.float32),
    mesh=plsc.VectorSubcoreMesh(core_axis_name="c", subcore_axis_name="s"),
    compiler_params=pltpu.CompilerParams(needs_layout_passes=False),
    scratch_types=[
        pltpu.VMEM((N_NODES,), jnp.float32),
        pltpu.VMEM((CHUNK,), jnp.int32),
        pltpu.VMEM((CHUNK,), jnp.int32),
        pltpu.VMEM((CHUNK,), jnp.float32),
        pltpu.VMEM((CHUNK,), jnp.float32),
        pltpu.VMEM((CHUNK,), jnp.float32),
        pltpu.VMEM((ACC_WORDS,), jnp.float32),
        pltpu.VMEM((NUM_SUBCORES, STRIPE), jnp.float32),
        pltpu.VMEM((STRIPE,), jnp.float32),
        pltpu.VMEM_SHARED((NUM_SUBCORES, ACC_WORDS), jnp.float32),
        pltpu.SemaphoreType.DMA,
        pltpu.SemaphoreType.DMA,
        pltpu.SemaphoreType.DMA,
        pltpu.SemaphoreType.DMA,
        pltpu.SemaphoreType.DMA,
        pltpu.SemaphoreType.DMA,
    ],
)(_sc_body)


def _finish_kernel(p_ref, o_ref):
    p = p_ref[...]
    tot = (p[0] + p[1]).reshape(ACC_WORDS // D_FEAT, D_FEAT)
    fidx = (lax.broadcasted_iota(jnp.int32, tot.shape, 0) * D_FEAT
            + lax.broadcasted_iota(jnp.int32, tot.shape, 1))
    nodes = jnp.where(fidx < N_NODES, tot, 0.0)
    kcl = jnp.sum(nodes * nodes) / jnp.float32(N_NODES)

    def _sum16(off):
        m = (fidx >= off) & (fidx < off + LANES)
        return jnp.sum(jnp.where(m, tot, 0.0))

    s_r, s_x = _sum16(OFF_SUM_R), _sum16(OFF_SUM_X)
    q_r, q_x = _sum16(OFF_SQ_R), _sum16(OFF_SQ_X)
    n = jnp.float32(N_EDGES)
    var_r = (q_r - s_r * s_r / n) / (n - 1.0)
    var_x = (q_x - s_x * s_x / n) / (n - 1.0)
    o_ref[0, 0] = kcl + 0.5 * (var_r + var_x)


def kernel(node_features, edge_index, edge_logits, edge_params):
    nf_flat = node_features.reshape(-1)
    r = edge_params[:, 0]
    x = edge_params[:, 1]
    v = _vx_kernel(nf_flat)
    partial = _sc_kernel(v, edge_index.reshape(-1), edge_logits, r, x)
    out = pl.pallas_call(
        _finish_kernel,
        out_shape=jax.ShapeDtypeStruct((1, 1), jnp.float32),
        out_specs=pl.BlockSpec(memory_space=pltpu.SMEM),
    )(partial)
    return out[0, 0]
